# baseline (speedup 1.0000x reference)
import functools

import jax
import jax.numpy as jnp
from jax import lax
from jax.experimental import pallas as pl

_MASK_RATIO = 0.5


def _mask_kernel(noise_ref, x_ref, out_ref, mask_ref, *, n, num_masked):
    a = noise_ref[:, 0, :]
    ai = a[:, :, None]
    aj = a[:, None, :]
    bc = a.shape[0]
    ii = lax.broadcasted_iota(jnp.int32, (bc, n, n), 1)
    jj = lax.broadcasted_iota(jnp.int32, (bc, n, n), 2)
    before = (aj < ai) | ((aj == ai) & (jj < ii))
    rank = jnp.sum(before.astype(jnp.int32), axis=2)
    masked = rank < num_masked
    mask_ref[...] = masked[:, None, :]
    out_ref[...] = jnp.where(masked[:, :, None, None], 0.0, x_ref[...])


def kernel(x):
    b, n, p, d = x.shape
    num_masked = int(_MASK_RATIO * n)
    bc = 4
    noise = jax.random.uniform(jax.random.key(1), (b, n), dtype=jnp.float32)
    noise3 = noise.reshape(b, 1, n)
    out, mask3 = pl.pallas_call(
        functools.partial(_mask_kernel, n=n, num_masked=num_masked),
        grid=(b // bc,),
        in_specs=[
            pl.BlockSpec((bc, 1, n), lambda i: (i, 0, 0)),
            pl.BlockSpec((bc, n, p, d), lambda i: (i, 0, 0, 0)),
        ],
        out_specs=[
            pl.BlockSpec((bc, n, p, d), lambda i: (i, 0, 0, 0)),
            pl.BlockSpec((bc, 1, n), lambda i: (i, 0, 0)),
        ],
        out_shape=[
            jax.ShapeDtypeStruct((b, n, p, d), x.dtype),
            jax.ShapeDtypeStruct((b, 1, n), jnp.bool_),
        ],
    )(noise3, x)
    return out, mask3.reshape(b, n)


# speedup vs baseline: 1.0003x; 1.0003x over previous
import functools

import jax
import jax.numpy as jnp
from jax import lax
from jax.experimental import pallas as pl

_MASK_RATIO = 0.5


def _mask_kernel(noise_ref, x_ref, out_ref, mask_ref, *, n, num_masked, bc):
    i = pl.program_id(0)
    a = noise_ref[pl.ds(i * bc, bc), 0, :]
    ai = a[:, :, None]
    aj = a[:, None, :]
    ii = lax.broadcasted_iota(jnp.int32, (bc, n, n), 1)
    jj = lax.broadcasted_iota(jnp.int32, (bc, n, n), 2)
    before = (aj < ai) | ((aj == ai) & (jj < ii))
    rank = jnp.sum(before.astype(jnp.int32), axis=2)
    masked = rank < num_masked
    mask_ref[pl.ds(i * bc, bc), :, :] = masked[:, None, :]
    out_ref[...] = jnp.where(masked[:, :, None, None], 0.0, x_ref[...])


def kernel(x):
    b, n, p, d = x.shape
    num_masked = int(_MASK_RATIO * n)
    bc = 4
    noise = jax.random.uniform(jax.random.key(1), (b, n), dtype=jnp.float32)
    noise3 = noise.reshape(b, 1, n)
    out, mask3 = pl.pallas_call(
        functools.partial(_mask_kernel, n=n, num_masked=num_masked, bc=bc),
        grid=(b // bc,),
        in_specs=[
            pl.BlockSpec((b, 1, n), lambda i: (0, 0, 0)),
            pl.BlockSpec((bc, n, p, d), lambda i: (i, 0, 0, 0)),
        ],
        out_specs=[
            pl.BlockSpec((bc, n, p, d), lambda i: (i, 0, 0, 0)),
            pl.BlockSpec((b, 1, n), lambda i: (0, 0, 0)),
        ],
        out_shape=[
            jax.ShapeDtypeStruct((b, n, p, d), x.dtype),
            jax.ShapeDtypeStruct((b, 1, n), jnp.bool_),
        ],
    )(noise3, x)
    return out, mask3.reshape(b, n)
